# Initial kernel scaffold; baseline (speedup 1.0000x reference)
import jax
import jax.numpy as jnp
from jax.experimental import pallas as pl


def kernel(inputs, boxes, box_indices):
    raise NotImplementedError("write your pallas kernel here")


# speedup vs baseline: 1.3330x; 1.3330x over previous
import functools

import jax
import jax.numpy as jnp
from jax import lax
from jax.experimental import pallas as pl
from jax.experimental.pallas import tpu as pltpu
from jax.experimental.pallas import tpu_sc as plsc

H = 224
W = 224
C = 192
N = 1000
NP = 1024
NW = 32
BPW = NP // NW
OH = 7
OW = 7
NCH = C // 16
ROWS = OW * 16


def _axis_params(v):
    t = v.astype(jnp.int32)
    fl = jnp.where(t.astype(jnp.float32) > v, t - 1, t)
    lerp = v - fl.astype(jnp.float32)
    valid = jnp.where((v >= 0.0) & (v <= 223.0), 1.0, 0.0).astype(jnp.float32)
    lo = jnp.clip(fl, 0, 223)
    hi = jnp.clip(fl + 1, 0, 223)
    wlo = valid * (1.0 - lerp)
    whi = valid * lerp
    return lo, hi, wlo, whi


def _roialign_body(img, boxes, bidx, out,
                   boxsm, bism, idxb, wbuf, rows, outrow, sem):
    c = lax.axis_index("c")
    s = lax.axis_index("s")
    wid = s * 2 + c
    lo = wid * BPW
    nb = jnp.minimum(BPW, jnp.maximum(0, N - lo))
    pltpu.sync_copy(boxes.at[pl.ds(lo * 4, BPW * 4)], boxsm.at[pl.ds(0, BPW * 4)])
    pltpu.sync_copy(bidx.at[pl.ds(lo, BPW)], bism.at[pl.ds(0, BPW)])

    lane = lax.iota(jnp.int32, 16)
    sy_hi = ((lane >> 3) & 1) == 1
    sx_hi = ((lane >> 2) & 1) == 1
    cy_hi = ((lane >> 1) & 1) == 1
    cx_hi = (lane & 1) == 1

    def box_loop(j, carry):
        bvec = boxsm[pl.ds(j * 4, 16)]
        bcy = bvec[0]
        bcx = bvec[1]
        bh = bvec[2]
        bw = bvec[3]
        y1 = bcy - bh * 0.5
        y2 = bcy + bh * 0.5
        x1 = bcx - bw * 0.5
        x2 = bcx + bw * 0.5
        bin_h = (y2 - y1) * (1.0 / 7.0)
        bin_w = (x2 - x1) * (1.0 / 7.0)
        gy1 = y1 + 0.25 * bin_h
        gy2 = y2 - 0.25 * bin_h
        gx1 = x1 + 0.25 * bin_w
        gx2 = x2 - 0.25 * bin_w
        hs = (gy2 - gy1) * ((H - 1.0) / 13.0)
        ws = (gx2 - gx1) * ((W - 1.0) / 13.0)
        y0f = gy1 * (H - 1.0)
        x0f = gx1 * (W - 1.0)
        base = bism[pl.ds(j, 16)][0] * (H * W)

        xpar = [_axis_params(x0f + float(ix) * ws) for ix in range(2 * OW)]

        def oy_loop(oy, carry2):
            oyf = (2 * oy).astype(jnp.float32)
            t0, b0, wt0, wb0 = _axis_params(y0f + oyf * hs)
            t1, b1, wt1, wb1 = _axis_params(y0f + (oyf + 1.0) * hs)
            y16 = jnp.where(cy_hi,
                            jnp.where(sy_hi, b1, b0),
                            jnp.where(sy_hi, t1, t0))
            wy16 = 0.25 * jnp.where(cy_hi,
                                    jnp.where(sy_hi, wb1, wb0),
                                    jnp.where(sy_hi, wt1, wt0))
            for ox in range(OW):
                l0, r0, wl0, wr0 = xpar[2 * ox]
                l1, r1, wl1, wr1 = xpar[2 * ox + 1]
                x16 = jnp.where(cx_hi,
                                jnp.where(sx_hi, r1, r0),
                                jnp.where(sx_hi, l1, l0))
                wx16 = jnp.where(cx_hi,
                                 jnp.where(sx_hi, wr1, wr0),
                                 jnp.where(sx_hi, wl1, wl0))
                idxb[pl.ds(ox * 16, 16)] = base + y16 * W + x16
                wbuf[pl.ds(ox * 16, 16)] = wy16 * wx16
            pltpu.async_copy(img.at[idxb], rows, sem).wait()

            def bin_loop(ox, carry3):
                rbase = ox * 16
                wvec = wbuf[pl.ds(rbase, 16)]
                accs = [jnp.zeros((16,), jnp.float32) for _ in range(NCH)]
                for k in range(16):
                    wk = wvec[k]
                    for ch in range(NCH):
                        accs[ch] = (accs[ch]
                                    + rows[rbase + k, pl.ds(ch * 16, 16)] * wk)
                for ch in range(NCH):
                    outrow[ox, pl.ds(ch * 16, 16)] = accs[ch]
                return carry3

            lax.fori_loop(0, OW, bin_loop, None)
            pltpu.sync_copy(outrow, out.at[lo + j, oy])
            return carry2

        lax.fori_loop(0, OH, oy_loop, None)
        return carry

    lax.fori_loop(0, nb, box_loop, None)


_roialign_sc = functools.partial(
    pl.kernel,
    out_type=jax.ShapeDtypeStruct((N, OH, OW, C), jnp.float32),
    mesh=plsc.VectorSubcoreMesh(core_axis_name="c", subcore_axis_name="s"),
    compiler_params=pltpu.CompilerParams(use_tc_tiling_on_sc=False),
    scratch_types=[
        pltpu.VMEM((BPW * 4 + 16,), jnp.float32),
        pltpu.VMEM((BPW + 16,), jnp.int32),
        pltpu.VMEM((ROWS,), jnp.int32),
        pltpu.VMEM((ROWS,), jnp.float32),
        pltpu.VMEM((ROWS, C), jnp.float32),
        pltpu.VMEM((OW, C), jnp.float32),
        pltpu.SemaphoreType.DMA,
    ],
)(_roialign_body)


def kernel(inputs, boxes, box_indices):
    img = inputs.reshape(8 * H * W, C)
    boxes_p = jnp.concatenate(
        [boxes, jnp.zeros((NP - N, 4), boxes.dtype)], axis=0).reshape(NP * 4)
    bidx_p = jnp.concatenate(
        [box_indices, jnp.zeros((NP - N,), box_indices.dtype)])
    return _roialign_sc(img, boxes_p, bidx_p)
